# baseline (speedup 1.0000x reference)
import jax
import jax.numpy as jnp
from jax.experimental import pallas as pl


def _cluster_kernel(data_ref, cent_ref, out_ref):
    data = data_ref[:, :]
    cent = cent_ref[:, :]
    xx = jnp.sum(data * data, axis=1)
    ccp1 = jnp.sum(cent * cent, axis=1) + 1.0
    g = jax.lax.dot_general(
        cent * -2.0, data, (((1,), (1,)), ((), ())),
        preferred_element_type=jnp.float32)
    u = g + ccp1[:, None] + xx[None, :]
    r = 1.0 / u
    t = r * r
    s = jnp.sum(t, axis=0)
    out_ref[:, :] = t * (1.0 / s)[None, :]


_BN = 512


def kernel(data, centroids):
    n, d = data.shape
    k, _ = centroids.shape
    return pl.pallas_call(
        _cluster_kernel,
        grid=(n // _BN,),
        in_specs=[
            pl.BlockSpec((_BN, d), lambda i: (i, 0)),
            pl.BlockSpec((k, d), lambda i: (0, 0)),
        ],
        out_specs=pl.BlockSpec((k, _BN), lambda i: (0, i)),
        out_shape=jax.ShapeDtypeStruct((k, n), jnp.float32),
    )(data, centroids)


# speedup vs baseline: 1.1097x; 1.1097x over previous
import jax
import jax.numpy as jnp
from jax.experimental import pallas as pl


def _cluster_kernel(data_ref, cent_ref, out_ref):
    data = data_ref[:, :]
    cent = cent_ref[:, :]
    xx = jnp.sum(data * data, axis=1)
    ccp1 = jnp.sum(cent * cent, axis=1) + 1.0
    g = jax.lax.dot_general(
        cent * -2.0, data, (((1,), (1,)), ((), ())),
        preferred_element_type=jnp.float32)
    u = g + ccp1[:, None] + xx[None, :]
    r = 1.0 / u
    t = r * r
    s = jnp.sum(t, axis=0)
    out_ref[:, :] = t * (1.0 / s)[None, :]


_BN = 1024


def kernel(data, centroids):
    n, d = data.shape
    k, _ = centroids.shape
    return pl.pallas_call(
        _cluster_kernel,
        grid=(n // _BN,),
        in_specs=[
            pl.BlockSpec((_BN, d), lambda i: (i, 0)),
            pl.BlockSpec((k, d), lambda i: (0, 0)),
        ],
        out_specs=pl.BlockSpec((k, _BN), lambda i: (0, i)),
        out_shape=jax.ShapeDtypeStruct((k, n), jnp.float32),
    )(data, centroids)


# speedup vs baseline: 1.1148x; 1.0046x over previous
import jax
import jax.numpy as jnp
from jax.experimental import pallas as pl

_BN = 1024


def _cluster_kernel(data_ref, cent_ref, out_ref):
    data = data_ref[:, :]
    cent = cent_ref[:, :]
    xx = jnp.sum(data * data, axis=1)
    ccp1 = jnp.sum(cent * cent, axis=1) + 1.0
    bn = data.shape[0]
    k = cent.shape[0]
    ca = jnp.concatenate(
        [cent * -2.0, ccp1[:, None], jnp.ones((k, 1), jnp.float32)], axis=1)
    xa = jnp.concatenate(
        [data, jnp.ones((bn, 1), jnp.float32), xx[:, None]], axis=1)
    u = jax.lax.dot_general(
        ca, xa, (((1,), (1,)), ((), ())),
        preferred_element_type=jnp.float32)
    t = 1.0 / (u * u)
    s = jnp.sum(t, axis=0)
    out_ref[:, :] = t * (1.0 / s)[None, :]


def kernel(data, centroids):
    n, d = data.shape
    k, _ = centroids.shape
    return pl.pallas_call(
        _cluster_kernel,
        grid=(n // _BN,),
        in_specs=[
            pl.BlockSpec((_BN, d), lambda i: (i, 0)),
            pl.BlockSpec((k, d), lambda i: (0, 0)),
        ],
        out_specs=pl.BlockSpec((k, _BN), lambda i: (0, i)),
        out_shape=jax.ShapeDtypeStruct((k, n), jnp.float32),
    )(data, centroids)
